# baseline (speedup 1.0000x reference)
import jax
import jax.numpy as jnp
from jax import lax
from jax.experimental import pallas as pl
from jax.experimental.pallas import tpu as pltpu
from jax.experimental.pallas import tpu_sc as plsc

N = 10000
E = 320000
D_IN = 128
D_E = 16
H = 64
L = 3
G = 64

NC = 2
NS = 16
NW = NC * NS
N8 = 10240
NR = N8 // 128
Ep = 327680
CE = 8192
NCH = Ep // CE
CER = CE // 128
ER = Ep // 128
GPC = CE // 16

BEB = 20480
NBE = Ep // BEB
BER = BEB // 128


def _sc_body(ht_hbm, ea_hbm, pidx_hbm, out_hbm,
             hcol0, hcol1, agg0, agg1, pb0, pb1, eb0, eb1,
             psem0, psem1, esem0, esem1):
    cid = lax.axis_index("c")
    sid = lax.axis_index("s")
    wid = sid * NC + cid
    hcols = (hcol0, hcol1)
    aggs = (agg0, agg1)
    bufs = ((pb0, eb0, psem0, esem0), (pb1, eb1, psem1, esem1))

    pltpu.sync_copy(ht_hbm.at[pl.ds((wid * 2) * N8, N8)], hcol0)
    pltpu.sync_copy(ht_hbm.at[pl.ds((wid * 2 + 1) * N8, N8)], hcol1)

    def _zrow(r, c):
        sl = pl.ds(r * 16, 16)
        agg0[sl] = jnp.zeros((16,), jnp.float32)
        agg1[sl] = jnp.zeros((16,), jnp.float32)
        return c

    lax.fori_loop(0, N8 // 16, _zrow, 0, unroll=8)

    def _issue(g, p):
        pb, eb, psem, esem = bufs[p]
        pltpu.async_copy(pidx_hbm.at[pl.ds(g * CE, CE)], pb, psem)
        pltpu.async_copy(ea_hbm.at[wid, :, pl.ds(g * CER, CER), :], eb, esem)

    def _process(g, p):
        pb, eb, psem, esem = bufs[p]
        pltpu.make_async_copy(pidx_hbm.at[pl.ds(g * CE, CE)], pb, psem).wait()
        pltpu.make_async_copy(
            ea_hbm.at[wid, :, pl.ds(g * CER, CER), :], eb, esem).wait()

        @plsc.parallel_loop(0, GPC, unroll=8)
        def _group(i):
            p16 = pb[pl.ds(i * 16, 16)]
            sv = lax.shift_right_logical(p16, 14)
            dv = jnp.bitwise_and(p16, 16383)
            erow = i // 8
            elane = (i % 8) * 16
            for c in range(2):
                hv = plsc.load_gather(hcols[c], [sv])
                eav = eb[c, erow, pl.ds(elane, 16)]
                m = jnp.maximum(hv + eav, 0.0)
                plsc.addupdate_scatter(aggs[c], [dv], m)

    _issue(0, 0)
    _issue(1, 1)

    def _pair(it, c):
        g0 = 2 * it
        _process(g0, 0)

        @pl.when(g0 + 2 < NCH)
        def _():
            _issue(g0 + 2, 0)

        _process(g0 + 1, 1)

        @pl.when(g0 + 3 < NCH)
        def _():
            _issue(g0 + 3, 1)

        return c

    lax.fori_loop(0, NCH // 2, _pair, 0)

    pltpu.sync_copy(agg0, out_hbm.at[pl.ds((wid * 2) * N8, N8)])
    pltpu.sync_copy(agg1, out_hbm.at[pl.ds((wid * 2 + 1) * N8, N8)])


_sc_layer = pl.kernel(
    _sc_body,
    out_type=jax.ShapeDtypeStruct((NW * 2 * N8,), jnp.float32),
    mesh=plsc.VectorSubcoreMesh(
        core_axis_name="c", subcore_axis_name="s",
        num_cores=NC, num_subcores=NS),
    compiler_params=pltpu.CompilerParams(needs_layout_passes=False),
    scratch_types=[
        pltpu.VMEM((N8,), jnp.float32),
        pltpu.VMEM((N8,), jnp.float32),
        pltpu.VMEM((N8,), jnp.float32),
        pltpu.VMEM((N8,), jnp.float32),
        pltpu.VMEM((CE,), jnp.int32),
        pltpu.VMEM((CE,), jnp.int32),
        pltpu.VMEM((2, CER, 128), jnp.float32),
        pltpu.VMEM((2, CER, 128), jnp.float32),
        pltpu.SemaphoreType.DMA,
        pltpu.SemaphoreType.DMA,
        pltpu.SemaphoreType.DMA,
        pltpu.SemaphoreType.DMA,
    ],
)


def _to_cols(hmat):
    hp = jnp.concatenate(
        [hmat, jnp.zeros((N8 - N, H), jnp.float32)], axis=0)
    return jnp.transpose(hp)


def _node_proj_body(x_ref, w_ref, b_ref, h_ref, ht_ref):
    h = (jnp.dot(x_ref[...], w_ref[...], preferred_element_type=jnp.float32)
         + b_ref[...])
    h_ref[...] = h
    ht_ref[...] = _to_cols(h)


_node_proj = pl.pallas_call(
    _node_proj_body,
    out_shape=[
        jax.ShapeDtypeStruct((N, H), jnp.float32),
        jax.ShapeDtypeStruct((H, N8), jnp.float32),
    ],
)


def _edge_proj_body(x_ref, w_ref, b_ref, o_ref):
    z = (jnp.dot(x_ref[...], w_ref[...], preferred_element_type=jnp.float32)
         + b_ref[...])
    o_ref[...] = jnp.transpose(z).reshape(NW, 2, BER, 128)


_edge_proj = pl.pallas_call(
    _edge_proj_body,
    grid=(NBE,),
    in_specs=[
        pl.BlockSpec((BEB, D_E), lambda i: (i, 0)),
        pl.BlockSpec((D_E, H), lambda i: (0, 0)),
        pl.BlockSpec((1, H), lambda i: (0, 0)),
    ],
    out_specs=pl.BlockSpec((NW, 2, BER, 128), lambda i: (0, 0, i, 0)),
    out_shape=jax.ShapeDtypeStruct((NW, 2, ER, 128), jnp.float32),
)


def _layer_core(h_ref, agg_ref, w1_ref, b1_ref, w2_ref, b2_ref, g_ref, b_ref):
    agg = jnp.transpose(agg_ref[...])[:N, :]
    z = h_ref[...] + agg
    z = jnp.maximum(
        jnp.dot(z, w1_ref[...], preferred_element_type=jnp.float32) + b1_ref[...],
        0.0,
    )
    z = jnp.dot(z, w2_ref[...], preferred_element_type=jnp.float32) + b2_ref[...]
    m = jnp.mean(z, axis=0, keepdims=True)
    zc = z - m
    var = jnp.mean(zc * zc, axis=0, keepdims=True)
    inv = g_ref[...] * lax.rsqrt(var + 1e-5)
    return jnp.maximum(zc * inv + b_ref[...], 0.0)


def _layer_body(h_ref, agg_ref, w1_ref, b1_ref, w2_ref, b2_ref, g_ref, b_ref,
                o_ref, ot_ref):
    hb = _layer_core(h_ref, agg_ref, w1_ref, b1_ref, w2_ref, b2_ref,
                     g_ref, b_ref)
    o_ref[...] = hb
    ot_ref[...] = _to_cols(hb)


_layer_tc = pl.pallas_call(
    _layer_body,
    out_shape=[
        jax.ShapeDtypeStruct((N, H), jnp.float32),
        jax.ShapeDtypeStruct((H, N8), jnp.float32),
    ],
)


def _layer_pool_body(h_ref, agg_ref, w1_ref, b1_ref, w2_ref, b2_ref,
                     g_ref, b_ref, bat_ref, o_ref, emb_ref):
    hb = _layer_core(h_ref, agg_ref, w1_ref, b1_ref, w2_ref, b2_ref,
                     g_ref, b_ref)
    o_ref[...] = hb
    ids = bat_ref[0, :]
    onehot = (ids[:, None]
              == lax.broadcasted_iota(jnp.int32, (N, G), 1)).astype(jnp.float32)
    sums = lax.dot_general(
        onehot, hb, (((0,), (0,)), ((), ())), preferred_element_type=jnp.float32)
    cnt = jnp.sum(onehot, axis=0)[:, None]
    emb_ref[...] = sums / jnp.maximum(cnt, 1.0)


_layer_pool_tc = pl.pallas_call(
    _layer_pool_body,
    out_shape=[
        jax.ShapeDtypeStruct((N, H), jnp.float32),
        jax.ShapeDtypeStruct((G, H), jnp.float32),
    ],
)


def kernel(x, edge_attr, node_W, node_b, edge_W, edge_b,
           mlp_W1, mlp_b1, mlp_W2, mlp_b2, bn_g, bn_b, edge_index, batch):
    pad = Ep - E
    srcp = jnp.concatenate([edge_index[0], jnp.zeros((pad,), jnp.int32)])
    dstp = jnp.concatenate(
        [edge_index[1], jnp.full((pad,), 10200, jnp.int32)])
    pidx = jnp.bitwise_or(jnp.left_shift(srcp, 14), dstp)
    eap = jnp.concatenate(
        [edge_attr, jnp.zeros((pad, D_E), jnp.float32)], axis=0)
    batch2 = batch.reshape(1, N)

    h, ht = _node_proj(x, node_W, node_b.reshape(1, H))
    ea4 = _edge_proj(eap, edge_W, edge_b.reshape(1, H))

    emb = None
    for l in range(L):
        agg1d = _sc_layer(ht.reshape(-1), ea4, pidx)
        agg4 = agg1d.reshape(H, N8)
        args = (h, agg4, mlp_W1[l], mlp_b1[l].reshape(1, H),
                mlp_W2[l], mlp_b2[l].reshape(1, H),
                bn_g[l].reshape(1, H), bn_b[l].reshape(1, H))
        if l < L - 1:
            h, ht = _layer_tc(*args)
        else:
            h, emb = _layer_pool_tc(*args, batch2)
    return (h, emb)


# speedup vs baseline: 1.0148x; 1.0148x over previous
import jax
import jax.numpy as jnp
from jax import lax
from jax.experimental import pallas as pl
from jax.experimental.pallas import tpu as pltpu
from jax.experimental.pallas import tpu_sc as plsc

N = 10000
E = 320000
D_IN = 128
D_E = 16
H = 64
L = 3
G = 64

NC = 2
NS = 16
NW = NC * NS
N8 = 10240
NR = N8 // 128
Ep = 327680
CE = 8192
NCH = Ep // CE
CER = CE // 128
ER = Ep // 128
GPC = CE // 16

BEB = 20480
NBE = Ep // BEB
BER = BEB // 128


def _sc_body(ht_hbm, ea_hbm, pidx_hbm, out_hbm,
             hcol0, hcol1, agg0, agg1, pb0, pb1, eb0, eb1,
             psem0, psem1, esem0, esem1):
    cid = lax.axis_index("c")
    sid = lax.axis_index("s")
    wid = sid * NC + cid
    hcols = (hcol0, hcol1)
    aggs = (agg0, agg1)
    bufs = ((pb0, eb0, psem0, esem0), (pb1, eb1, psem1, esem1))

    pltpu.sync_copy(ht_hbm.at[pl.ds((wid * 2) * N8, N8)], hcol0)
    pltpu.sync_copy(ht_hbm.at[pl.ds((wid * 2 + 1) * N8, N8)], hcol1)

    def _zrow(r, c):
        sl = pl.ds(r * 16, 16)
        agg0[sl] = jnp.zeros((16,), jnp.float32)
        agg1[sl] = jnp.zeros((16,), jnp.float32)
        return c

    lax.fori_loop(0, N8 // 16, _zrow, 0, unroll=8)

    def _issue(g, p):
        pb, eb, psem, esem = bufs[p]
        pltpu.async_copy(pidx_hbm.at[pl.ds(g * CE, CE)], pb, psem)
        pltpu.async_copy(ea_hbm.at[wid, :, pl.ds(g * CER, CER), :], eb, esem)

    def _process(g, p):
        pb, eb, psem, esem = bufs[p]
        pltpu.make_async_copy(pidx_hbm.at[pl.ds(g * CE, CE)], pb, psem).wait()
        pltpu.make_async_copy(
            ea_hbm.at[wid, :, pl.ds(g * CER, CER), :], eb, esem).wait()

        @plsc.parallel_loop(0, GPC, unroll=4)
        def _group(i):
            p16 = pb[pl.ds(i * 16, 16)]
            sv = lax.shift_right_logical(p16, 14)
            dv = jnp.bitwise_and(p16, 16383)
            erow = i // 8
            elane = (i % 8) * 16
            for c in range(2):
                hv = plsc.load_gather(hcols[c], [sv])
                eav = eb[c, erow, pl.ds(elane, 16)]
                m = jnp.maximum(hv + eav, 0.0)
                plsc.addupdate_scatter(aggs[c], [dv], m)

    _issue(0, 0)
    _issue(1, 1)

    def _pair(it, c):
        g0 = 2 * it
        _process(g0, 0)

        @pl.when(g0 + 2 < NCH)
        def _():
            _issue(g0 + 2, 0)

        _process(g0 + 1, 1)

        @pl.when(g0 + 3 < NCH)
        def _():
            _issue(g0 + 3, 1)

        return c

    lax.fori_loop(0, NCH // 2, _pair, 0)

    pltpu.sync_copy(agg0, out_hbm.at[pl.ds((wid * 2) * N8, N8)])
    pltpu.sync_copy(agg1, out_hbm.at[pl.ds((wid * 2 + 1) * N8, N8)])


_sc_layer = pl.kernel(
    _sc_body,
    out_type=jax.ShapeDtypeStruct((NW * 2 * N8,), jnp.float32),
    mesh=plsc.VectorSubcoreMesh(
        core_axis_name="c", subcore_axis_name="s",
        num_cores=NC, num_subcores=NS),
    compiler_params=pltpu.CompilerParams(needs_layout_passes=False),
    scratch_types=[
        pltpu.VMEM((N8,), jnp.float32),
        pltpu.VMEM((N8,), jnp.float32),
        pltpu.VMEM((N8,), jnp.float32),
        pltpu.VMEM((N8,), jnp.float32),
        pltpu.VMEM((CE,), jnp.int32),
        pltpu.VMEM((CE,), jnp.int32),
        pltpu.VMEM((2, CER, 128), jnp.float32),
        pltpu.VMEM((2, CER, 128), jnp.float32),
        pltpu.SemaphoreType.DMA,
        pltpu.SemaphoreType.DMA,
        pltpu.SemaphoreType.DMA,
        pltpu.SemaphoreType.DMA,
    ],
)


def _to_cols(hmat):
    hp = jnp.concatenate(
        [hmat, jnp.zeros((N8 - N, H), jnp.float32)], axis=0)
    return jnp.transpose(hp)


def _node_proj_body(x_ref, w_ref, b_ref, h_ref, ht_ref):
    h = (jnp.dot(x_ref[...], w_ref[...], preferred_element_type=jnp.float32)
         + b_ref[...])
    h_ref[...] = h
    ht_ref[...] = _to_cols(h)


_node_proj = pl.pallas_call(
    _node_proj_body,
    out_shape=[
        jax.ShapeDtypeStruct((N, H), jnp.float32),
        jax.ShapeDtypeStruct((H, N8), jnp.float32),
    ],
)


def _edge_proj_body(x_ref, w_ref, b_ref, o_ref):
    z = (jnp.dot(x_ref[...], w_ref[...], preferred_element_type=jnp.float32)
         + b_ref[...])
    o_ref[...] = jnp.transpose(z).reshape(NW, 2, BER, 128)


_edge_proj = pl.pallas_call(
    _edge_proj_body,
    grid=(NBE,),
    in_specs=[
        pl.BlockSpec((BEB, D_E), lambda i: (i, 0)),
        pl.BlockSpec((D_E, H), lambda i: (0, 0)),
        pl.BlockSpec((1, H), lambda i: (0, 0)),
    ],
    out_specs=pl.BlockSpec((NW, 2, BER, 128), lambda i: (0, 0, i, 0)),
    out_shape=jax.ShapeDtypeStruct((NW, 2, ER, 128), jnp.float32),
)


def _layer_core(h_ref, agg_ref, w1_ref, b1_ref, w2_ref, b2_ref, g_ref, b_ref):
    agg = jnp.transpose(agg_ref[...])[:N, :]
    z = h_ref[...] + agg
    z = jnp.maximum(
        jnp.dot(z, w1_ref[...], preferred_element_type=jnp.float32) + b1_ref[...],
        0.0,
    )
    z = jnp.dot(z, w2_ref[...], preferred_element_type=jnp.float32) + b2_ref[...]
    m = jnp.mean(z, axis=0, keepdims=True)
    zc = z - m
    var = jnp.mean(zc * zc, axis=0, keepdims=True)
    inv = g_ref[...] * lax.rsqrt(var + 1e-5)
    return jnp.maximum(zc * inv + b_ref[...], 0.0)


def _layer_body(h_ref, agg_ref, w1_ref, b1_ref, w2_ref, b2_ref, g_ref, b_ref,
                o_ref, ot_ref):
    hb = _layer_core(h_ref, agg_ref, w1_ref, b1_ref, w2_ref, b2_ref,
                     g_ref, b_ref)
    o_ref[...] = hb
    ot_ref[...] = _to_cols(hb)


_layer_tc = pl.pallas_call(
    _layer_body,
    out_shape=[
        jax.ShapeDtypeStruct((N, H), jnp.float32),
        jax.ShapeDtypeStruct((H, N8), jnp.float32),
    ],
)


def _layer_pool_body(h_ref, agg_ref, w1_ref, b1_ref, w2_ref, b2_ref,
                     g_ref, b_ref, bat_ref, o_ref, emb_ref):
    hb = _layer_core(h_ref, agg_ref, w1_ref, b1_ref, w2_ref, b2_ref,
                     g_ref, b_ref)
    o_ref[...] = hb
    ids = bat_ref[0, :]
    onehot = (ids[:, None]
              == lax.broadcasted_iota(jnp.int32, (N, G), 1)).astype(jnp.float32)
    sums = lax.dot_general(
        onehot, hb, (((0,), (0,)), ((), ())), preferred_element_type=jnp.float32)
    cnt = jnp.sum(onehot, axis=0)[:, None]
    emb_ref[...] = sums / jnp.maximum(cnt, 1.0)


_layer_pool_tc = pl.pallas_call(
    _layer_pool_body,
    out_shape=[
        jax.ShapeDtypeStruct((N, H), jnp.float32),
        jax.ShapeDtypeStruct((G, H), jnp.float32),
    ],
)


def kernel(x, edge_attr, node_W, node_b, edge_W, edge_b,
           mlp_W1, mlp_b1, mlp_W2, mlp_b2, bn_g, bn_b, edge_index, batch):
    pad = Ep - E
    srcp = jnp.concatenate([edge_index[0], jnp.zeros((pad,), jnp.int32)])
    dstp = jnp.concatenate(
        [edge_index[1], jnp.full((pad,), 10200, jnp.int32)])
    pidx = jnp.bitwise_or(jnp.left_shift(srcp, 14), dstp)
    eap = jnp.concatenate(
        [edge_attr, jnp.zeros((pad, D_E), jnp.float32)], axis=0)
    batch2 = batch.reshape(1, N)

    h, ht = _node_proj(x, node_W, node_b.reshape(1, H))
    ea4 = _edge_proj(eap, edge_W, edge_b.reshape(1, H))

    emb = None
    for l in range(L):
        agg1d = _sc_layer(ht.reshape(-1), ea4, pidx)
        agg4 = agg1d.reshape(H, N8)
        args = (h, agg4, mlp_W1[l], mlp_b1[l].reshape(1, H),
                mlp_W2[l], mlp_b2[l].reshape(1, H),
                bn_g[l].reshape(1, H), bn_b[l].reshape(1, H))
        if l < L - 1:
            h, ht = _layer_tc(*args)
        else:
            h, emb = _layer_pool_tc(*args, batch2)
    return (h, emb)


# speedup vs baseline: 1.0428x; 1.0276x over previous
import jax
import jax.numpy as jnp
from jax import lax
from jax.experimental import pallas as pl
from jax.experimental.pallas import tpu as pltpu
from jax.experimental.pallas import tpu_sc as plsc

N = 10000
E = 320000
D_IN = 128
D_E = 16
H = 64
L = 3
G = 64

NC = 2
NS = 16
NW = NC * NS
N8 = 10240
NR = N8 // 128
Ep = 327680
CE = 8192
NCH = Ep // CE
CER = CE // 128
ER = Ep // 128
GPC = CE // 16

BEB = 20480
NBE = Ep // BEB
BER = BEB // 128


def _sc_body(ht_hbm, ea_hbm, pidx_hbm, out_hbm,
             hcol0, hcol1, agg0, agg1, pb0, pb1, eb0, eb1,
             psem0, psem1, esem0, esem1):
    cid = lax.axis_index("c")
    sid = lax.axis_index("s")
    wid = sid * NC + cid
    hcols = (hcol0, hcol1)
    aggs = (agg0, agg1)
    bufs = ((pb0, eb0, psem0, esem0), (pb1, eb1, psem1, esem1))

    pltpu.sync_copy(ht_hbm.at[pl.ds((wid * 2) * N8, N8)], hcol0)
    pltpu.sync_copy(ht_hbm.at[pl.ds((wid * 2 + 1) * N8, N8)], hcol1)

    def _zrow(r, c):
        sl = pl.ds(r * 16, 16)
        agg0[sl] = jnp.zeros((16,), jnp.float32)
        agg1[sl] = jnp.zeros((16,), jnp.float32)
        return c

    lax.fori_loop(0, N8 // 16, _zrow, 0, unroll=8)

    def _issue(g, p):
        pb, eb, psem, esem = bufs[p]
        pltpu.async_copy(pidx_hbm.at[pl.ds(g * CE, CE)], pb, psem)
        pltpu.async_copy(ea_hbm.at[wid, :, pl.ds(g * CER, CER), :], eb, esem)

    def _process(g, p):
        pb, eb, psem, esem = bufs[p]
        pltpu.make_async_copy(pidx_hbm.at[pl.ds(g * CE, CE)], pb, psem).wait()
        pltpu.make_async_copy(
            ea_hbm.at[wid, :, pl.ds(g * CER, CER), :], eb, esem).wait()

        @plsc.parallel_loop(0, GPC, unroll=4)
        def _group(i):
            p16 = pb[pl.ds(i * 16, 16)]
            sv = lax.shift_right_logical(p16, 14)
            dv = jnp.bitwise_and(p16, 16383)
            erow = i // 8
            elane = (i % 8) * 16
            for c in range(2):
                hv = plsc.load_gather(hcols[c], [sv])
                eav = eb[c, erow, pl.ds(elane, 16)]
                m = jnp.maximum(hv + eav, 0.0)
                plsc.addupdate_scatter(aggs[c], [dv], m)

    _issue(0, 0)
    _issue(1, 1)

    def _pair(it, c):
        g0 = 2 * it
        _process(g0, 0)

        @pl.when(g0 + 2 < NCH)
        def _():
            _issue(g0 + 2, 0)

        _process(g0 + 1, 1)

        @pl.when(g0 + 3 < NCH)
        def _():
            _issue(g0 + 3, 1)

        return c

    lax.fori_loop(0, NCH // 2, _pair, 0)

    pltpu.sync_copy(agg0, out_hbm.at[pl.ds((wid * 2) * N8, N8)])
    pltpu.sync_copy(agg1, out_hbm.at[pl.ds((wid * 2 + 1) * N8, N8)])


_sc_layer = pl.kernel(
    _sc_body,
    out_type=jax.ShapeDtypeStruct((NW * 2 * N8,), jnp.float32),
    mesh=plsc.VectorSubcoreMesh(
        core_axis_name="c", subcore_axis_name="s",
        num_cores=NC, num_subcores=NS),
    compiler_params=pltpu.CompilerParams(needs_layout_passes=False),
    scratch_types=[
        pltpu.VMEM((N8,), jnp.float32),
        pltpu.VMEM((N8,), jnp.float32),
        pltpu.VMEM((N8,), jnp.float32),
        pltpu.VMEM((N8,), jnp.float32),
        pltpu.VMEM((CE,), jnp.int32),
        pltpu.VMEM((CE,), jnp.int32),
        pltpu.VMEM((2, CER, 128), jnp.float32),
        pltpu.VMEM((2, CER, 128), jnp.float32),
        pltpu.SemaphoreType.DMA,
        pltpu.SemaphoreType.DMA,
        pltpu.SemaphoreType.DMA,
        pltpu.SemaphoreType.DMA,
    ],
)


def _to_cols(hmat):
    hp = jnp.concatenate(
        [hmat, jnp.zeros((N8 - N, H), jnp.float32)], axis=0)
    return jnp.transpose(hp).reshape(H * N8)


def _node_proj_body(x_ref, w_ref, b_ref, h_ref, ht_ref):
    h = (jnp.dot(x_ref[...], w_ref[...], preferred_element_type=jnp.float32)
         + b_ref[...])
    h_ref[...] = h
    ht_ref[...] = _to_cols(h)


_node_proj = pl.pallas_call(
    _node_proj_body,
    out_shape=[
        jax.ShapeDtypeStruct((N, H), jnp.float32),
        jax.ShapeDtypeStruct((H * N8,), jnp.float32),
    ],
)


def _edge_proj_body(x_ref, w_ref, b_ref, o_ref):
    z = (jnp.dot(x_ref[...], w_ref[...], preferred_element_type=jnp.float32)
         + b_ref[...])
    o_ref[...] = jnp.transpose(z).reshape(NW, 2, BER, 128)


_edge_proj = pl.pallas_call(
    _edge_proj_body,
    grid=(NBE,),
    in_specs=[
        pl.BlockSpec((BEB, D_E), lambda i: (i, 0)),
        pl.BlockSpec((D_E, H), lambda i: (0, 0)),
        pl.BlockSpec((1, H), lambda i: (0, 0)),
    ],
    out_specs=pl.BlockSpec((NW, 2, BER, 128), lambda i: (0, 0, i, 0)),
    out_shape=jax.ShapeDtypeStruct((NW, 2, ER, 128), jnp.float32),
)


def _layer_core(h_ref, agg_ref, w1_ref, b1_ref, w2_ref, b2_ref, g_ref, b_ref):
    agg = jnp.transpose(agg_ref[...].reshape(H, N8))[:N, :]
    z = h_ref[...] + agg
    z = jnp.maximum(
        jnp.dot(z, w1_ref[...], preferred_element_type=jnp.float32) + b1_ref[...],
        0.0,
    )
    z = jnp.dot(z, w2_ref[...], preferred_element_type=jnp.float32) + b2_ref[...]
    m = jnp.mean(z, axis=0, keepdims=True)
    zc = z - m
    var = jnp.mean(zc * zc, axis=0, keepdims=True)
    inv = g_ref[...] * lax.rsqrt(var + 1e-5)
    return jnp.maximum(zc * inv + b_ref[...], 0.0)


def _layer_body(h_ref, agg_ref, w1_ref, b1_ref, w2_ref, b2_ref, g_ref, b_ref,
                o_ref, ot_ref):
    hb = _layer_core(h_ref, agg_ref, w1_ref, b1_ref, w2_ref, b2_ref,
                     g_ref, b_ref)
    o_ref[...] = hb
    ot_ref[...] = _to_cols(hb)


_layer_tc = pl.pallas_call(
    _layer_body,
    out_shape=[
        jax.ShapeDtypeStruct((N, H), jnp.float32),
        jax.ShapeDtypeStruct((H * N8,), jnp.float32),
    ],
)


def _layer_pool_body(h_ref, agg_ref, w1_ref, b1_ref, w2_ref, b2_ref,
                     g_ref, b_ref, bat_ref, o_ref, emb_ref):
    hb = _layer_core(h_ref, agg_ref, w1_ref, b1_ref, w2_ref, b2_ref,
                     g_ref, b_ref)
    o_ref[...] = hb
    ids = bat_ref[0, :]
    onehot = (ids[:, None]
              == lax.broadcasted_iota(jnp.int32, (N, G), 1)).astype(jnp.float32)
    sums = lax.dot_general(
        onehot, hb, (((0,), (0,)), ((), ())), preferred_element_type=jnp.float32)
    cnt = jnp.sum(onehot, axis=0)[:, None]
    emb_ref[...] = sums / jnp.maximum(cnt, 1.0)


_layer_pool_tc = pl.pallas_call(
    _layer_pool_body,
    out_shape=[
        jax.ShapeDtypeStruct((N, H), jnp.float32),
        jax.ShapeDtypeStruct((G, H), jnp.float32),
    ],
)


def kernel(x, edge_attr, node_W, node_b, edge_W, edge_b,
           mlp_W1, mlp_b1, mlp_W2, mlp_b2, bn_g, bn_b, edge_index, batch):
    pad = Ep - E
    srcp = jnp.concatenate([edge_index[0], jnp.zeros((pad,), jnp.int32)])
    dstp = jnp.concatenate(
        [edge_index[1], jnp.full((pad,), 10200, jnp.int32)])
    pidx = jnp.bitwise_or(jnp.left_shift(srcp, 14), dstp)
    eap = jnp.concatenate(
        [edge_attr, jnp.zeros((pad, D_E), jnp.float32)], axis=0)
    batch2 = batch.reshape(1, N)

    h, ht = _node_proj(x, node_W, node_b.reshape(1, H))
    ea4 = _edge_proj(eap, edge_W, edge_b.reshape(1, H))

    emb = None
    for l in range(L):
        agg1d = _sc_layer(ht, ea4, pidx)
        args = (h, agg1d, mlp_W1[l], mlp_b1[l].reshape(1, H),
                mlp_W2[l], mlp_b2[l].reshape(1, H),
                bn_g[l].reshape(1, H), bn_b[l].reshape(1, H))
        if l < L - 1:
            h, ht = _layer_tc(*args)
        else:
            h, emb = _layer_pool_tc(*args, batch2)
    return (h, emb)
